# baseline (speedup 1.0000x reference)
import functools

import jax
import jax.numpy as jnp
from jax import lax
from jax.experimental import pallas as pl
from jax.experimental.pallas import tpu as pltpu
from jax.experimental.pallas import tpu_sc as plsc

_INFO = plsc.get_sparse_core_info()
_NC = _INFO.num_cores
_NS = _INFO.num_subcores
_NW = _NC * _NS


@functools.lru_cache(maxsize=None)
def _make_gather(batch: int, dim: int):
  b_per_w = batch // _NW
  mesh = plsc.VectorSubcoreMesh(core_axis_name="c", subcore_axis_name="s")

  @functools.partial(
      pl.kernel,
      mesh=mesh,
      out_type=jax.ShapeDtypeStruct((batch, dim), jnp.float32),
      scratch_types=[
          pltpu.VMEM((b_per_w,), jnp.int32),
          pltpu.VMEM((b_per_w, dim), jnp.float32),
          pltpu.SemaphoreType.DMA,
      ],
      compiler_params=pltpu.CompilerParams(use_tc_tiling_on_sc=False),
  )
  def gather_kernel(ids_hbm, table_hbm, out_hbm, idx_v, rows_v, sem):
    wid = lax.axis_index("s") * _NC + lax.axis_index("c")
    base = wid * b_per_w
    pltpu.sync_copy(ids_hbm.at[pl.ds(base, b_per_w)], idx_v)
    pltpu.async_copy(table_hbm.at[idx_v], rows_v, sem).wait()
    pltpu.sync_copy(rows_v, out_hbm.at[pl.ds(base, b_per_w)])

  return gather_kernel


@jax.jit
def kernel(ids, table):
  batch, = ids.shape
  dim = table.shape[1]
  return _make_gather(batch, dim)(ids, table)


# speedup vs baseline: 1.0013x; 1.0013x over previous
import functools

import jax
import jax.numpy as jnp
from jax import lax
from jax.experimental import pallas as pl
from jax.experimental.pallas import tpu as pltpu
from jax.experimental.pallas import tpu_sc as plsc

_INFO = plsc.get_sparse_core_info()
_NC = _INFO.num_cores
_NS = _INFO.num_subcores
_NW = _NC * _NS


@functools.lru_cache(maxsize=None)
def _make_gather(batch: int, dim: int):
  b_per_w = batch // _NW
  mesh = plsc.VectorSubcoreMesh(core_axis_name="c", subcore_axis_name="s")

  @functools.partial(
      pl.kernel,
      mesh=mesh,
      out_type=jax.ShapeDtypeStruct((batch, dim), jnp.float32),
      scratch_types=[
          pltpu.VMEM((b_per_w,), jnp.int32),
          pltpu.VMEM((b_per_w, dim), jnp.float32),
          pltpu.SemaphoreType.DMA,
      ],
      compiler_params=pltpu.CompilerParams(
          use_tc_tiling_on_sc=False, needs_layout_passes=False
      ),
  )
  def gather_kernel(ids_hbm, table_hbm, out_hbm, idx_v, rows_v, sem):
    wid = lax.axis_index("s") * _NC + lax.axis_index("c")
    base = wid * b_per_w
    pltpu.sync_copy(ids_hbm.at[pl.ds(base, b_per_w)], idx_v)
    pltpu.async_copy(table_hbm.at[idx_v], rows_v, sem).wait()
    pltpu.sync_copy(rows_v, out_hbm.at[pl.ds(base, b_per_w)])

  return gather_kernel


@jax.jit
def kernel(ids, table):
  batch, = ids.shape
  dim = table.shape[1]
  return _make_gather(batch, dim)(ids, table)


# speedup vs baseline: 4.2096x; 4.2043x over previous
import functools

import jax
import jax.numpy as jnp
from jax import lax
from jax.experimental import pallas as pl
from jax.experimental.pallas import tpu as pltpu
from jax.experimental.pallas import tpu_sc as plsc

_INFO = plsc.get_sparse_core_info()
_NC = _INFO.num_cores
_NS = _INFO.num_subcores
_NW = _NC * _NS
_LANES = 128
_GRP = 16


@functools.lru_cache(maxsize=None)
def _make_gather(batch: int, dim: int):
  b_per_w = batch // _NW
  n_groups = b_per_w // _GRP
  mesh = plsc.VectorSubcoreMesh(core_axis_name="c", subcore_axis_name="s")
  n_dim_vecs = dim // 16

  @functools.partial(
      pl.kernel,
      mesh=mesh,
      out_type=jax.ShapeDtypeStruct((dim, batch), jnp.float32),
      scratch_types=[
          pltpu.VMEM((b_per_w + _GRP,), jnp.int32),
          pltpu.VMEM((_GRP, dim, _LANES), jnp.float32),
          pltpu.VMEM((dim, b_per_w), jnp.float32),
          pltpu.SemaphoreType.DMA((_GRP,)),
      ],
      compiler_params=pltpu.CompilerParams(needs_layout_passes=False),
  )
  def gather_kernel(ids_hbm, table_t_hbm, out_hbm, ids_v, bufs, cols_v, sems):
    wid = lax.axis_index("s") * _NC + lax.axis_index("c")
    base = wid * b_per_w
    pltpu.sync_copy(ids_hbm.at[pl.ds(base, b_per_w)], ids_v.at[pl.ds(0, b_per_w)])

    def start_fetch(i_scalar, slot):
      col0 = pl.multiple_of((i_scalar // _LANES) * _LANES, _LANES)
      pltpu.async_copy(
          table_t_hbm.at[:, pl.ds(col0, _LANES)], bufs.at[slot], sems.at[slot]
      )

    vec0 = ids_v[pl.ds(0, _GRP)]
    for j in range(_GRP):
      start_fetch(vec0[j], j)

    dim_base = jax.lax.broadcasted_iota(jnp.int32, (16,), 0)

    def body(g, carry):
      k0 = g * _GRP
      vec = ids_v[pl.ds(k0, _GRP)]
      vec_next = ids_v[pl.ds(k0 + _GRP, _GRP)]
      for j in range(_GRP):
        pltpu.make_async_copy(
            table_t_hbm.at[:, pl.ds(0, _LANES)], bufs.at[j], sems.at[j]
        ).wait()
        lane = jnp.full((16,), lax.rem(vec[j], _LANES), dtype=jnp.int32)
        kvec = jnp.full((16,), k0 + j, dtype=jnp.int32)
        for q in range(n_dim_vecs):
          dims = dim_base + (16 * q)
          vals = plsc.load_gather(bufs.at[j], [dims, lane])
          plsc.store_scatter(cols_v, [dims, kvec], vals)

        @pl.when(g + 1 < n_groups)
        def _():
          start_fetch(vec_next[j], j)

      return carry

    lax.fori_loop(0, n_groups, body, 0)
    pltpu.sync_copy(cols_v, out_hbm.at[:, pl.ds(base, b_per_w)])

  return gather_kernel


@jax.jit
def kernel(ids, table):
  batch, = ids.shape
  dim = table.shape[1]
  out_t = _make_gather(batch, dim)(ids, table.T)
  return out_t.T
